# baseline (speedup 1.0000x reference)
import jax
import jax.numpy as jnp
from jax.experimental import pallas as pl
from jax.experimental.pallas import tpu as pltpu


def kernel(x, shape, row_embed, col_embed):
    b, _, h, w = x.shape
    f = row_embed.shape[1]
    hw = h * w

    b_blk = 8

    def body(shape_ref, col_ref, row_ref, out_ref):
        residual = (shape_ref[2] * shape_ref[3] - hw).astype(jnp.float32)
        col = col_ref[:w, :]
        row = row_ref[:h, :]
        colt = jnp.broadcast_to(col[None, :, :], (h, w, f)).reshape(hw, f)
        rowt = jnp.broadcast_to(row[:, None, :], (h, w, f)).reshape(hw, f)
        out_ref[:, :, :f] = jnp.broadcast_to(colt[None] + residual, (b_blk, hw, f))
        out_ref[:, :, f:] = jnp.broadcast_to(rowt[None] + residual, (b_blk, hw, f))

    grid_spec = pltpu.PrefetchScalarGridSpec(
        num_scalar_prefetch=1,
        grid=(b // b_blk,),
        in_specs=[
            pl.BlockSpec(col_embed.shape, lambda i, s: (0, 0)),
            pl.BlockSpec(row_embed.shape, lambda i, s: (0, 0)),
        ],
        out_specs=pl.BlockSpec((b_blk, hw, 2 * f), lambda i, s: (i, 0, 0)),
    )

    return pl.pallas_call(
        body,
        grid_spec=grid_spec,
        out_shape=jax.ShapeDtypeStruct((b, hw, 2 * f), jnp.float32),
    )(shape, col_embed, row_embed)


# speedup vs baseline: 1.0310x; 1.0310x over previous
import jax
import jax.numpy as jnp
from jax.experimental import pallas as pl
from jax.experimental.pallas import tpu as pltpu


def kernel(x, shape, row_embed, col_embed):
    b, _, h, w = x.shape
    f = row_embed.shape[1]
    hw = h * w

    b_blk = 2

    def body(shape_ref, col_ref, row_ref, out_ref):
        residual = (shape_ref[2] * shape_ref[3] - hw).astype(jnp.float32)
        col = col_ref[:w, :]
        row = row_ref[:h, :]
        colt = jnp.broadcast_to(col[None, :, :], (h, w, f)).reshape(hw, f)
        rowt = jnp.broadcast_to(row[:, None, :], (h, w, f)).reshape(hw, f)
        out_ref[:, :, :f] = jnp.broadcast_to(colt[None] + residual, (b_blk, hw, f))
        out_ref[:, :, f:] = jnp.broadcast_to(rowt[None] + residual, (b_blk, hw, f))

    grid_spec = pltpu.PrefetchScalarGridSpec(
        num_scalar_prefetch=1,
        grid=(b // b_blk,),
        in_specs=[
            pl.BlockSpec(col_embed.shape, lambda i, s: (0, 0)),
            pl.BlockSpec(row_embed.shape, lambda i, s: (0, 0)),
        ],
        out_specs=pl.BlockSpec((b_blk, hw, 2 * f), lambda i, s: (i, 0, 0)),
    )

    return pl.pallas_call(
        body,
        grid_spec=grid_spec,
        out_shape=jax.ShapeDtypeStruct((b, hw, 2 * f), jnp.float32),
    )(shape, col_embed, row_embed)


# speedup vs baseline: 1.1709x; 1.1357x over previous
import jax
import jax.numpy as jnp
from jax.experimental import pallas as pl
from jax.experimental.pallas import tpu as pltpu


def kernel(x, shape, row_embed, col_embed):
    b, _, h, w = x.shape
    f = row_embed.shape[1]
    hw = h * w

    def body(shape_ref, col_ref, row_ref, out_ref, pos_ref, sem):
        residual = (shape_ref[2] * shape_ref[3] - hw).astype(jnp.float32)
        col = col_ref[:w, :]
        row = row_ref[:h, :]
        pos_ref[:, :f] = jnp.broadcast_to(col[None], (h, w, f)).reshape(hw, f) + residual
        pos_ref[:, f:] = jnp.broadcast_to(row[:, None], (h, w, f)).reshape(hw, f) + residual
        copies = [
            pltpu.make_async_copy(pos_ref, out_ref.at[i], sem.at[i])
            for i in range(b)
        ]
        for c in copies:
            c.start()
        for c in copies:
            c.wait()

    grid_spec = pltpu.PrefetchScalarGridSpec(
        num_scalar_prefetch=1,
        grid=(1,),
        in_specs=[
            pl.BlockSpec(col_embed.shape, lambda i, s: (0, 0)),
            pl.BlockSpec(row_embed.shape, lambda i, s: (0, 0)),
        ],
        out_specs=pl.BlockSpec(memory_space=pl.ANY),
        scratch_shapes=[
            pltpu.VMEM((hw, 2 * f), jnp.float32),
            pltpu.SemaphoreType.DMA((b,)),
        ],
    )

    return pl.pallas_call(
        body,
        grid_spec=grid_spec,
        out_shape=jax.ShapeDtypeStruct((b, hw, 2 * f), jnp.float32),
    )(shape, col_embed, row_embed)
